# baseline (speedup 1.0000x reference)
import functools

import jax
import jax.numpy as jnp
from jax import lax
from jax.experimental import pallas as pl
from jax.experimental.pallas import tpu as pltpu
from jax.experimental.pallas import tpu_sc as plsc

NC, NS, L = 2, 16, 16
NW = NC * NS

NA = 10000
NB = 160000
MAXNB = 16
H = 600
HP = 640
FA = 133
FB = 147
DEPTH = 6

NA_PAD = 10240
A_PER_W = NA_PAD // NW
SEG_CA = 4
B_PER_W = NB // NW
GD_CB = 40

_mesh = plsc.VectorSubcoreMesh(core_axis_name="c", subcore_axis_name="s")


SEG_NCH = A_PER_W // SEG_CA
GD_NCH = B_PER_W // GD_CB


@functools.partial(
    pl.kernel,
    out_type=jax.ShapeDtypeStruct((NA_PAD, HP), jnp.float32),
    mesh=_mesh,
    scratch_types=[
        pltpu.VMEM((A_PER_W * MAXNB,), jnp.int32),
        pltpu.VMEM((SEG_CA * MAXNB, HP), jnp.float32),
        pltpu.VMEM((SEG_CA * MAXNB, HP), jnp.float32),
        pltpu.VMEM((SEG_CA, HP), jnp.float32),
        pltpu.VMEM((SEG_CA, HP), jnp.float32),
        pltpu.SemaphoreType.DMA,
        pltpu.SemaphoreType.DMA,
        pltpu.SemaphoreType.DMA,
        pltpu.SemaphoreType.DMA,
    ],
)
def _sc_segsum(msg_hbm, idx_hbm, out_hbm, idx_all, rows0, rows1,
               acc0, acc1, semg0, semg1, sems0, sems1):
    wid = lax.axis_index("s") * NC + lax.axis_index("c")
    rows = (rows0, rows1)
    accs, semgs, semss = (acc0, acc1), (semg0, semg1), (sems0, sems1)
    pltpu.sync_copy(idx_hbm.at[pl.ds(wid * A_PER_W * MAXNB, A_PER_W * MAXNB)],
                    idx_all)

    def idx_at(ci):
        return idx_all.at[pl.ds(ci * SEG_CA * MAXNB, SEG_CA * MAXNB)]

    def fire(ci, p):
        pltpu.async_copy(msg_hbm.at[idx_at(ci)], rows[p], semgs[p])

    def out_at(ci):
        return out_hbm.at[pl.ds(wid * A_PER_W + ci * SEG_CA, SEG_CA)]

    def consume(ci, p):
        pltpu.make_async_copy(msg_hbm.at[idx_at(ci)], rows[p], semgs[p]).wait()

        @pl.when(ci >= 2)
        def _():
            wait_store(ci - 2, p)

        rp, ac = rows[p], accs[p]

        def atom_body(a, carry):
            base = a * MAXNB
            for c in range(HP // L):
                vals = [rp[base + j, pl.ds(c * L, L)] for j in range(MAXNB)]
                while len(vals) > 1:
                    vals = [vals[i] + vals[i + 1]
                            for i in range(0, len(vals), 2)]
                ac[a, pl.ds(c * L, L)] = vals[0]
            return carry

        lax.fori_loop(0, SEG_CA, atom_body, 0)
        pltpu.async_copy(ac, out_at(ci), semss[p])

    def wait_store(ci, p):
        pltpu.make_async_copy(accs[p], out_at(ci), semss[p]).wait()

    fire(0, 0)
    fire(1, 1)

    def body(cj, carry):
        ci = 2 * cj
        consume(ci, 0)
        fire(ci + 2, 0)
        consume(ci + 1, 1)
        fire(ci + 3, 1)
        return carry

    lax.fori_loop(0, SEG_NCH // 2 - 1, body, 0)
    consume(SEG_NCH - 2, 0)
    consume(SEG_NCH - 1, 1)
    wait_store(SEG_NCH - 2, 0)
    wait_store(SEG_NCH - 1, 1)


@functools.partial(
    pl.kernel,
    out_type=jax.ShapeDtypeStruct((NB, HP), jnp.float32),
    mesh=_mesh,
    scratch_types=[
        pltpu.VMEM((GD_NCH * 2 * GD_CB,), jnp.int32),
        pltpu.VMEM((GD_CB, HP), jnp.float32),
        pltpu.VMEM((GD_CB, HP), jnp.float32),
        pltpu.VMEM((GD_CB, HP), jnp.float32),
        pltpu.VMEM((GD_CB, HP), jnp.float32),
        pltpu.SemaphoreType.DMA,
        pltpu.SemaphoreType.DMA,
        pltpu.SemaphoreType.DMA,
        pltpu.SemaphoreType.DMA,
        pltpu.SemaphoreType.DMA,
        pltpu.SemaphoreType.DMA,
    ],
)
def _sc_gather_diff(am_hbm, msg_hbm, idx2_hbm, out_hbm,
                    ii_all, arow0, arow1, mrow0, mrow1,
                    semA0, semA1, semM0, semM1, semS0, semS1):
    wid = lax.axis_index("s") * NC + lax.axis_index("c")
    arows, mrows = (arow0, arow1), (mrow0, mrow1)
    semsA, semsM = (semA0, semA1), (semM0, semM1)
    semsS = (semS0, semS1)
    pltpu.sync_copy(
        idx2_hbm.at[pl.ds(wid * GD_NCH * 2 * GD_CB, GD_NCH * 2 * GD_CB)],
        ii_all)

    def idxA_at(ci):
        return ii_all.at[pl.ds(ci * 2 * GD_CB, GD_CB)]

    def idxM_at(ci):
        return ii_all.at[pl.ds(ci * 2 * GD_CB + GD_CB, GD_CB)]

    def fire_a(ci, p):
        pltpu.async_copy(am_hbm.at[idxA_at(ci)], arows[p], semsA[p])

    def fire_m(ci, p):
        pltpu.async_copy(msg_hbm.at[idxM_at(ci)], mrows[p], semsM[p])

    def out_at(ci):
        return out_hbm.at[pl.ds(wid * B_PER_W + ci * GD_CB, GD_CB)]

    def wait_store(ci, p):
        pltpu.make_async_copy(mrows[p], out_at(ci), semsS[p]).wait()

    def compute_store(ci, p):
        pltpu.make_async_copy(am_hbm.at[idxA_at(ci)],
                              arows[p], semsA[p]).wait()
        pltpu.make_async_copy(msg_hbm.at[idxM_at(ci)],
                              mrows[p], semsM[p]).wait()
        ap, mp = arows[p], mrows[p]

        def row_body(r, carry2):
            for c in range(HP // L):
                mp[r, pl.ds(c * L, L)] = (
                    ap[r, pl.ds(c * L, L)] - mp[r, pl.ds(c * L, L)])
            return carry2

        lax.fori_loop(0, GD_CB, row_body, 0)
        pltpu.async_copy(mp, out_at(ci), semsS[p])

    fire_a(0, 0)
    fire_a(1, 1)
    fire_m(0, 0)
    compute_store(0, 0)
    fire_a(2, 0)
    fire_m(1, 1)
    compute_store(1, 1)
    fire_a(3, 1)
    wait_store(0, 0)
    fire_m(2, 0)

    def body(cj, carry):
        s0 = 2 * cj
        compute_store(s0, 0)
        fire_a(s0 + 2, 0)
        wait_store(s0 - 1, 1)
        fire_m(s0 + 1, 1)
        compute_store(s0 + 1, 1)
        fire_a(s0 + 3, 1)
        wait_store(s0, 0)
        fire_m(s0 + 2, 0)
        return carry

    lax.fori_loop(1, (GD_NCH - 1) // 2 - 1, body, 0)
    compute_store(GD_NCH - 3, 0)
    fire_a(GD_NCH - 1, 0)
    wait_store(GD_NCH - 4, 1)
    fire_m(GD_NCH - 2, 1)
    compute_store(GD_NCH - 2, 1)
    wait_store(GD_NCH - 3, 0)
    fire_m(GD_NCH - 1, 0)
    compute_store(GD_NCH - 1, 0)
    wait_store(GD_NCH - 2, 1)
    wait_store(GD_NCH - 1, 0)


BM = 800
BMA = 400


def _mm_in_body(fb_ref, w_ref, inp_ref, msg_ref):
    x = jnp.dot(fb_ref[...].astype(jnp.bfloat16), w_ref[...],
                preferred_element_type=jnp.float32)
    inp_ref[...] = x
    msg_ref[...] = jnp.maximum(x, 0.0)


def _tc_in(f_bonds, w_i_t):
    return pl.pallas_call(
        _mm_in_body,
        grid=(NB // BM,),
        in_specs=[
            pl.BlockSpec((BM, FB), lambda i: (i, 0)),
            pl.BlockSpec((FB, HP), lambda i: (0, 0)),
        ],
        out_specs=[
            pl.BlockSpec((BM, HP), lambda i: (i, 0)),
            pl.BlockSpec((BM, HP), lambda i: (i, 0)),
        ],
        out_shape=[
            jax.ShapeDtypeStruct((NB, HP), jnp.float32),
            jax.ShapeDtypeStruct((NB, HP), jnp.float32),
        ],
    )(f_bonds, w_i_t)


def _mm_h_body(t_ref, w_ref, inp_ref, out_ref):
    x = jnp.dot(t_ref[...].astype(jnp.bfloat16), w_ref[...],
                preferred_element_type=jnp.float32)
    out_ref[...] = jnp.maximum(inp_ref[...] + x, 0.0)


def _tc_h(t, w_h_t, inp):
    return pl.pallas_call(
        _mm_h_body,
        grid=(NB // BM,),
        in_specs=[
            pl.BlockSpec((BM, HP), lambda i: (i, 0)),
            pl.BlockSpec((HP, HP), lambda i: (0, 0)),
            pl.BlockSpec((BM, HP), lambda i: (i, 0)),
        ],
        out_specs=pl.BlockSpec((BM, HP), lambda i: (i, 0)),
        out_shape=jax.ShapeDtypeStruct((NB, HP), jnp.float32),
    )(t, w_h_t, inp)


def _readout_body(fa_ref, am_ref, woa_ref, woh_ref, bo_ref,
                  w1_ref, b1_ref, w2_ref, b2_ref, wr_ref, br_ref, out_ref):
    x = jnp.dot(fa_ref[...], woa_ref[...], preferred_element_type=jnp.float32)
    x = x + jnp.dot(am_ref[...][:, :H], woh_ref[...],
                    preferred_element_type=jnp.float32)
    x = jnp.maximum(x + bo_ref[...], 0.0)
    h = jnp.maximum(
        jnp.dot(x, w1_ref[...], preferred_element_type=jnp.float32)
        + b1_ref[...], 0.0)
    h = jnp.maximum(
        jnp.dot(h, w2_ref[...], preferred_element_type=jnp.float32)
        + b2_ref[...], 0.0)
    o = jnp.dot(h, wr_ref[...], preferred_element_type=jnp.float32) + br_ref[...]
    out_ref[...] = jax.nn.sigmoid(o)


def _tc_readout(f_atoms, am, woa_t, woh_t, b_o, w1_t, b1, w2_t, b2, wr_t, br):
    return pl.pallas_call(
        _readout_body,
        grid=(NA // BMA,),
        in_specs=[
            pl.BlockSpec((BMA, FA), lambda i: (i, 0)),
            pl.BlockSpec((BMA, HP), lambda i: (i, 0)),
            pl.BlockSpec((FA, H), lambda i: (0, 0)),
            pl.BlockSpec((H, H), lambda i: (0, 0)),
            pl.BlockSpec((1, H), lambda i: (0, 0)),
            pl.BlockSpec((H, H), lambda i: (0, 0)),
            pl.BlockSpec((1, H), lambda i: (0, 0)),
            pl.BlockSpec((H, H), lambda i: (0, 0)),
            pl.BlockSpec((1, H), lambda i: (0, 0)),
            pl.BlockSpec((H, 1), lambda i: (0, 0)),
            pl.BlockSpec((1, 1), lambda i: (0, 0)),
        ],
        out_specs=pl.BlockSpec((BMA, 1), lambda i: (i, 0)),
        out_shape=jax.ShapeDtypeStruct((NA, 1), jnp.float32),
    )(f_atoms, am, woa_t, woh_t, b_o, w1_t, b1, w2_t, b2, wr_t, br)


def kernel(f_atoms, f_bonds, a2b, b2a, b2revb,
           W_i, W_h, W_o, b_o, W_f1, b_f1, W_f2, b_f2, W_r, b_r):
    w_i_t = jnp.zeros((FB, HP), jnp.bfloat16).at[:, :H].set(
        W_i.T.astype(jnp.bfloat16))
    w_h_t = jnp.zeros((HP, HP), jnp.bfloat16).at[:H, :H].set(
        W_h.T.astype(jnp.bfloat16))
    a2b_flat = jnp.pad(a2b, ((0, NA_PAD - NA), (0, 0))).reshape(-1)
    idx2 = jnp.stack([b2a.reshape(NB // GD_CB, GD_CB),
                      b2revb.reshape(NB // GD_CB, GD_CB)], axis=1).reshape(-1)

    inp, msg = _tc_in(f_bonds, w_i_t)
    for _ in range(DEPTH - 1):
        am = _sc_segsum(msg, a2b_flat)
        t = _sc_gather_diff(am, msg, idx2)
        msg = _tc_h(t, w_h_t, inp)
    am = _sc_segsum(msg, a2b_flat)

    out = _tc_readout(
        f_atoms, am,
        W_o[:, :FA].T, W_o[:, FA:].T, b_o.reshape(1, H),
        W_f1.T, b_f1.reshape(1, H), W_f2.T, b_f2.reshape(1, H),
        W_r.T, b_r.reshape(1, 1))
    return out[1:]


# speedup vs baseline: 1.1246x; 1.1246x over previous
import functools

import jax
import jax.numpy as jnp
from jax import lax
from jax.experimental import pallas as pl
from jax.experimental.pallas import tpu as pltpu
from jax.experimental.pallas import tpu_sc as plsc

NC, NS, L = 2, 16, 16
NW = NC * NS

NA = 10000
NB = 160000
MAXNB = 16
H = 600
HP = 640
FA = 133
FB = 147
DEPTH = 6

NA_PAD = 10240
A_PER_W = NA_PAD // NW
SEG_CA = 4
B_PER_W = NB // NW
GD_CB = 40

_mesh = plsc.VectorSubcoreMesh(core_axis_name="c", subcore_axis_name="s")


SEG_NCH = A_PER_W // SEG_CA
GD_NCH = B_PER_W // GD_CB


@functools.partial(
    pl.kernel,
    out_type=jax.ShapeDtypeStruct((NA_PAD, HP), jnp.float32),
    mesh=_mesh,
    scratch_types=[
        pltpu.VMEM((A_PER_W * MAXNB,), jnp.int32),
        pltpu.VMEM((SEG_CA * MAXNB, HP), jnp.float32),
        pltpu.VMEM((SEG_CA * MAXNB, HP), jnp.float32),
        pltpu.VMEM((SEG_CA, HP), jnp.float32),
        pltpu.VMEM((SEG_CA, HP), jnp.float32),
        pltpu.SemaphoreType.DMA,
        pltpu.SemaphoreType.DMA,
        pltpu.SemaphoreType.DMA,
        pltpu.SemaphoreType.DMA,
    ],
)
def _sc_segsum(msg_hbm, idx_hbm, out_hbm, idx_all, rows0, rows1,
               acc0, acc1, semg0, semg1, sems0, sems1):
    wid = lax.axis_index("s") * NC + lax.axis_index("c")
    rows = (rows0, rows1)
    accs, semgs, semss = (acc0, acc1), (semg0, semg1), (sems0, sems1)
    pltpu.sync_copy(idx_hbm.at[pl.ds(wid * A_PER_W * MAXNB, A_PER_W * MAXNB)],
                    idx_all)

    def idx_at(ci):
        return idx_all.at[pl.ds(ci * SEG_CA * MAXNB, SEG_CA * MAXNB)]

    def fire(ci, p):
        pltpu.async_copy(msg_hbm.at[idx_at(ci)], rows[p], semgs[p])

    def out_at(ci):
        return out_hbm.at[pl.ds(wid * A_PER_W + ci * SEG_CA, SEG_CA)]

    def consume(ci, p):
        pltpu.make_async_copy(msg_hbm.at[idx_at(ci)], rows[p], semgs[p]).wait()

        @pl.when(ci >= 2)
        def _():
            wait_store(ci - 2, p)

        rp, ac = rows[p], accs[p]

        def atom_body(a, carry):
            base = a * MAXNB
            for c in range(HP // L):
                vals = [rp[base + j, pl.ds(c * L, L)] for j in range(MAXNB)]
                while len(vals) > 1:
                    vals = [vals[i] + vals[i + 1]
                            for i in range(0, len(vals), 2)]
                ac[a, pl.ds(c * L, L)] = vals[0]
            return carry

        lax.fori_loop(0, SEG_CA, atom_body, 0)
        pltpu.async_copy(ac, out_at(ci), semss[p])

    def wait_store(ci, p):
        pltpu.make_async_copy(accs[p], out_at(ci), semss[p]).wait()

    fire(0, 0)
    fire(1, 1)

    def body(cj, carry):
        ci = 2 * cj
        consume(ci, 0)
        fire(ci + 2, 0)
        consume(ci + 1, 1)
        fire(ci + 3, 1)
        return carry

    lax.fori_loop(0, SEG_NCH // 2 - 1, body, 0)
    consume(SEG_NCH - 2, 0)
    consume(SEG_NCH - 1, 1)
    wait_store(SEG_NCH - 2, 0)
    wait_store(SEG_NCH - 1, 1)


@functools.partial(
    pl.kernel,
    out_type=jax.ShapeDtypeStruct((NB, HP), jnp.float32),
    mesh=_mesh,
    scratch_types=[
        pltpu.VMEM((GD_NCH * 2 * GD_CB,), jnp.int32),
        pltpu.VMEM((GD_CB, HP), jnp.float32),
        pltpu.VMEM((GD_CB, HP), jnp.float32),
        pltpu.VMEM((GD_CB, HP), jnp.float32),
        pltpu.VMEM((GD_CB, HP), jnp.float32),
        pltpu.SemaphoreType.DMA,
        pltpu.SemaphoreType.DMA,
        pltpu.SemaphoreType.DMA,
        pltpu.SemaphoreType.DMA,
        pltpu.SemaphoreType.DMA,
        pltpu.SemaphoreType.DMA,
    ],
)
def _sc_gather_diff(am_hbm, msg_hbm, idx2_hbm, out_hbm,
                    ii_all, arow0, arow1, mrow0, mrow1,
                    semA0, semA1, semM0, semM1, semS0, semS1):
    wid = lax.axis_index("s") * NC + lax.axis_index("c")
    arows, mrows = (arow0, arow1), (mrow0, mrow1)
    semsA, semsM = (semA0, semA1), (semM0, semM1)
    semsS = (semS0, semS1)
    pltpu.sync_copy(
        idx2_hbm.at[pl.ds(wid * GD_NCH * 2 * GD_CB, GD_NCH * 2 * GD_CB)],
        ii_all)

    def idxA_at(ci):
        return ii_all.at[pl.ds(ci * 2 * GD_CB, GD_CB)]

    def idxM_at(ci):
        return ii_all.at[pl.ds(ci * 2 * GD_CB + GD_CB, GD_CB)]

    def fire(ci, p):
        pltpu.async_copy(am_hbm.at[idxA_at(ci)], arows[p], semsA[p])
        pltpu.async_copy(msg_hbm.at[idxM_at(ci)], mrows[p], semsM[p])

    def out_at(ci):
        return out_hbm.at[pl.ds(wid * B_PER_W + ci * GD_CB, GD_CB)]

    def consume(ci, p):
        pltpu.make_async_copy(am_hbm.at[idxA_at(ci)],
                              arows[p], semsA[p]).wait()
        pltpu.make_async_copy(msg_hbm.at[idxM_at(ci)],
                              mrows[p], semsM[p]).wait()
        ap, mp = arows[p], mrows[p]

        def row_body(r, carry2):
            for c in range(HP // L):
                ap[r, pl.ds(c * L, L)] = (
                    ap[r, pl.ds(c * L, L)] - mp[r, pl.ds(c * L, L)])
            return carry2

        lax.fori_loop(0, GD_CB, row_body, 0)
        pltpu.sync_copy(ap, out_at(ci))

    fire(0, 0)
    fire(1, 1)

    def body(cj, carry):
        ci = 2 * cj
        consume(ci, 0)
        fire(ci + 2, 0)
        consume(ci + 1, 1)
        fire(ci + 3, 1)
        return carry

    lax.fori_loop(0, (GD_NCH - 1) // 2 - 1, body, 0)
    consume(GD_NCH - 3, 0)
    fire(GD_NCH - 1, 0)
    consume(GD_NCH - 2, 1)
    consume(GD_NCH - 1, 0)


BM = 800
BMA = 400


def _mm_in_body(fb_ref, w_ref, inp_ref, msg_ref):
    x = jnp.dot(fb_ref[...].astype(jnp.bfloat16), w_ref[...],
                preferred_element_type=jnp.float32)
    inp_ref[...] = x
    msg_ref[...] = jnp.maximum(x, 0.0)


def _tc_in(f_bonds, w_i_t):
    return pl.pallas_call(
        _mm_in_body,
        grid=(NB // BM,),
        in_specs=[
            pl.BlockSpec((BM, FB), lambda i: (i, 0)),
            pl.BlockSpec((FB, HP), lambda i: (0, 0)),
        ],
        out_specs=[
            pl.BlockSpec((BM, HP), lambda i: (i, 0)),
            pl.BlockSpec((BM, HP), lambda i: (i, 0)),
        ],
        out_shape=[
            jax.ShapeDtypeStruct((NB, HP), jnp.float32),
            jax.ShapeDtypeStruct((NB, HP), jnp.float32),
        ],
    )(f_bonds, w_i_t)


def _mm_h_body(t_ref, w_ref, inp_ref, out_ref):
    x = jnp.dot(t_ref[...].astype(jnp.bfloat16), w_ref[...],
                preferred_element_type=jnp.float32)
    out_ref[...] = jnp.maximum(inp_ref[...] + x, 0.0)


def _tc_h(t, w_h_t, inp):
    return pl.pallas_call(
        _mm_h_body,
        grid=(NB // BM,),
        in_specs=[
            pl.BlockSpec((BM, HP), lambda i: (i, 0)),
            pl.BlockSpec((HP, HP), lambda i: (0, 0)),
            pl.BlockSpec((BM, HP), lambda i: (i, 0)),
        ],
        out_specs=pl.BlockSpec((BM, HP), lambda i: (i, 0)),
        out_shape=jax.ShapeDtypeStruct((NB, HP), jnp.float32),
    )(t, w_h_t, inp)


def _readout_body(fa_ref, am_ref, woa_ref, woh_ref, bo_ref,
                  w1_ref, b1_ref, w2_ref, b2_ref, wr_ref, br_ref, out_ref):
    x = jnp.dot(fa_ref[...], woa_ref[...], preferred_element_type=jnp.float32)
    x = x + jnp.dot(am_ref[...][:, :H], woh_ref[...],
                    preferred_element_type=jnp.float32)
    x = jnp.maximum(x + bo_ref[...], 0.0)
    h = jnp.maximum(
        jnp.dot(x, w1_ref[...], preferred_element_type=jnp.float32)
        + b1_ref[...], 0.0)
    h = jnp.maximum(
        jnp.dot(h, w2_ref[...], preferred_element_type=jnp.float32)
        + b2_ref[...], 0.0)
    o = jnp.dot(h, wr_ref[...], preferred_element_type=jnp.float32) + br_ref[...]
    out_ref[...] = jax.nn.sigmoid(o)


def _tc_readout(f_atoms, am, woa_t, woh_t, b_o, w1_t, b1, w2_t, b2, wr_t, br):
    return pl.pallas_call(
        _readout_body,
        grid=(NA // BMA,),
        in_specs=[
            pl.BlockSpec((BMA, FA), lambda i: (i, 0)),
            pl.BlockSpec((BMA, HP), lambda i: (i, 0)),
            pl.BlockSpec((FA, H), lambda i: (0, 0)),
            pl.BlockSpec((H, H), lambda i: (0, 0)),
            pl.BlockSpec((1, H), lambda i: (0, 0)),
            pl.BlockSpec((H, H), lambda i: (0, 0)),
            pl.BlockSpec((1, H), lambda i: (0, 0)),
            pl.BlockSpec((H, H), lambda i: (0, 0)),
            pl.BlockSpec((1, H), lambda i: (0, 0)),
            pl.BlockSpec((H, 1), lambda i: (0, 0)),
            pl.BlockSpec((1, 1), lambda i: (0, 0)),
        ],
        out_specs=pl.BlockSpec((BMA, 1), lambda i: (i, 0)),
        out_shape=jax.ShapeDtypeStruct((NA, 1), jnp.float32),
    )(f_atoms, am, woa_t, woh_t, b_o, w1_t, b1, w2_t, b2, wr_t, br)


def kernel(f_atoms, f_bonds, a2b, b2a, b2revb,
           W_i, W_h, W_o, b_o, W_f1, b_f1, W_f2, b_f2, W_r, b_r):
    w_i_t = jnp.zeros((FB, HP), jnp.bfloat16).at[:, :H].set(
        W_i.T.astype(jnp.bfloat16))
    w_h_t = jnp.zeros((HP, HP), jnp.bfloat16).at[:H, :H].set(
        W_h.T.astype(jnp.bfloat16))
    a2b_flat = jnp.pad(a2b, ((0, NA_PAD - NA), (0, 0))).reshape(-1)
    idx2 = jnp.stack([b2a.reshape(NB // GD_CB, GD_CB),
                      b2revb.reshape(NB // GD_CB, GD_CB)], axis=1).reshape(-1)

    inp, msg = _tc_in(f_bonds, w_i_t)
    for _ in range(DEPTH - 1):
        am = _sc_segsum(msg, a2b_flat)
        t = _sc_gather_diff(am, msg, idx2)
        msg = _tc_h(t, w_h_t, inp)
    am = _sc_segsum(msg, a2b_flat)

    out = _tc_readout(
        f_atoms, am,
        W_o[:, :FA].T, W_o[:, FA:].T, b_o.reshape(1, H),
        W_f1.T, b_f1.reshape(1, H), W_f2.T, b_f2.reshape(1, H),
        W_r.T, b_r.reshape(1, 1))
    return out[1:]


# speedup vs baseline: 1.1583x; 1.0300x over previous
import functools

import jax
import jax.numpy as jnp
from jax import lax
from jax.experimental import pallas as pl
from jax.experimental.pallas import tpu as pltpu
from jax.experimental.pallas import tpu_sc as plsc

NC, NS, L = 2, 16, 16
NW = NC * NS

NA = 10000
NB = 160000
MAXNB = 16
H = 600
HP = 640
FA = 133
FB = 147
DEPTH = 6

NA_PAD = 10240
A_PER_W = NA_PAD // NW
SEG_CA = 4
B_PER_W = NB // NW
GD_CB = 40

_mesh = plsc.VectorSubcoreMesh(core_axis_name="c", subcore_axis_name="s")


SEG_NCH = A_PER_W // SEG_CA
GD_NCH = B_PER_W // GD_CB


@functools.partial(
    pl.kernel,
    out_type=jax.ShapeDtypeStruct((NA_PAD, HP), jnp.float32),
    mesh=_mesh,
    scratch_types=[
        pltpu.VMEM((A_PER_W * MAXNB,), jnp.int32),
        pltpu.VMEM((SEG_CA * MAXNB, HP), jnp.float32),
        pltpu.VMEM((SEG_CA * MAXNB, HP), jnp.float32),
        pltpu.VMEM((SEG_CA, HP), jnp.float32),
        pltpu.VMEM((SEG_CA, HP), jnp.float32),
        pltpu.SemaphoreType.DMA,
        pltpu.SemaphoreType.DMA,
        pltpu.SemaphoreType.DMA,
        pltpu.SemaphoreType.DMA,
    ],
)
def _sc_segsum(msg_hbm, idx_hbm, out_hbm, idx_all, rows0, rows1,
               acc0, acc1, semg0, semg1, sems0, sems1):
    wid = lax.axis_index("s") * NC + lax.axis_index("c")
    rows = (rows0, rows1)
    accs, semgs, semss = (acc0, acc1), (semg0, semg1), (sems0, sems1)
    pltpu.sync_copy(idx_hbm.at[pl.ds(wid * A_PER_W * MAXNB, A_PER_W * MAXNB)],
                    idx_all)

    def idx_at(ci):
        return idx_all.at[pl.ds(ci * SEG_CA * MAXNB, SEG_CA * MAXNB)]

    def fire(ci, p):
        pltpu.async_copy(msg_hbm.at[idx_at(ci)], rows[p], semgs[p])

    def out_at(ci):
        return out_hbm.at[pl.ds(wid * A_PER_W + ci * SEG_CA, SEG_CA)]

    def consume(ci, p):
        pltpu.make_async_copy(msg_hbm.at[idx_at(ci)], rows[p], semgs[p]).wait()

        @pl.when(ci >= 2)
        def _():
            wait_store(ci - 2, p)

        rp, ac = rows[p], accs[p]

        def atom_body(a, carry):
            base = a * MAXNB
            for c in range(HP // L):
                vals = [rp[base + j, pl.ds(c * L, L)] for j in range(MAXNB)]
                while len(vals) > 1:
                    vals = [vals[i] + vals[i + 1]
                            for i in range(0, len(vals), 2)]
                ac[a, pl.ds(c * L, L)] = vals[0]
            return carry

        lax.fori_loop(0, SEG_CA, atom_body, 0)
        pltpu.async_copy(ac, out_at(ci), semss[p])

    def wait_store(ci, p):
        pltpu.make_async_copy(accs[p], out_at(ci), semss[p]).wait()

    fire(0, 0)
    fire(1, 1)

    def body(cj, carry):
        ci = 2 * cj
        consume(ci, 0)
        fire(ci + 2, 0)
        consume(ci + 1, 1)
        fire(ci + 3, 1)
        return carry

    lax.fori_loop(0, SEG_NCH // 2 - 1, body, 0)
    consume(SEG_NCH - 2, 0)
    consume(SEG_NCH - 1, 1)
    wait_store(SEG_NCH - 2, 0)
    wait_store(SEG_NCH - 1, 1)


@functools.partial(
    pl.kernel,
    out_type=jax.ShapeDtypeStruct((NB, HP), jnp.float32),
    mesh=_mesh,
    scratch_types=[
        pltpu.VMEM((GD_NCH * 2 * GD_CB,), jnp.int32),
        pltpu.VMEM((GD_CB, HP), jnp.float32),
        pltpu.VMEM((GD_CB, HP), jnp.float32),
        pltpu.VMEM((GD_CB, HP), jnp.float32),
        pltpu.VMEM((GD_CB, HP), jnp.float32),
        pltpu.SemaphoreType.DMA,
        pltpu.SemaphoreType.DMA,
        pltpu.SemaphoreType.DMA,
        pltpu.SemaphoreType.DMA,
        pltpu.SemaphoreType.DMA,
        pltpu.SemaphoreType.DMA,
    ],
)
def _sc_gather_diff(am_hbm, msg_hbm, idx2_hbm, out_hbm,
                    ii_all, arow0, arow1, mrow0, mrow1,
                    semA0, semA1, semM0, semM1, semS0, semS1):
    wid = lax.axis_index("s") * NC + lax.axis_index("c")
    arows, mrows = (arow0, arow1), (mrow0, mrow1)
    semsA, semsM = (semA0, semA1), (semM0, semM1)
    semsS = (semS0, semS1)
    pltpu.sync_copy(
        idx2_hbm.at[pl.ds(wid * GD_NCH * 2 * GD_CB, GD_NCH * 2 * GD_CB)],
        ii_all)

    def idxA_at(ci):
        return ii_all.at[pl.ds(ci * 2 * GD_CB, GD_CB)]

    def idxM_at(ci):
        return ii_all.at[pl.ds(ci * 2 * GD_CB + GD_CB, GD_CB)]

    def fire(ci, p):
        pltpu.async_copy(am_hbm.at[idxA_at(ci)], arows[p], semsA[p])
        pltpu.async_copy(msg_hbm.at[idxM_at(ci)], mrows[p], semsM[p])

    def out_at(ci):
        return out_hbm.at[pl.ds(wid * B_PER_W + ci * GD_CB, GD_CB)]

    def consume(ci, p):
        pltpu.make_async_copy(am_hbm.at[idxA_at(ci)],
                              arows[p], semsA[p]).wait()
        pltpu.make_async_copy(msg_hbm.at[idxM_at(ci)],
                              mrows[p], semsM[p]).wait()
        ap, mp = arows[p], mrows[p]

        def row_body(r, carry2):
            for c in range(HP // L):
                ap[r, pl.ds(c * L, L)] = (
                    ap[r, pl.ds(c * L, L)] - mp[r, pl.ds(c * L, L)])
            return carry2

        lax.fori_loop(0, GD_CB, row_body, 0)
        pltpu.sync_copy(ap, out_at(ci))

    fire(0, 0)
    fire(1, 1)

    def body(cj, carry):
        ci = 2 * cj
        consume(ci, 0)
        fire(ci + 2, 0)
        consume(ci + 1, 1)
        fire(ci + 3, 1)
        return carry

    lax.fori_loop(0, (GD_NCH - 1) // 2 - 1, body, 0)
    consume(GD_NCH - 3, 0)
    fire(GD_NCH - 1, 0)
    consume(GD_NCH - 2, 1)
    consume(GD_NCH - 1, 0)


BM = 800
BMA = 400


def _mm_in_body(fb_ref, w_ref, inp_ref, msg_ref):
    x = jnp.dot(fb_ref[...].astype(jnp.bfloat16), w_ref[...],
                preferred_element_type=jnp.float32)
    inp_ref[...] = x.astype(jnp.bfloat16)
    msg_ref[...] = jnp.maximum(x, 0.0)


def _tc_in(f_bonds, w_i_t):
    return pl.pallas_call(
        _mm_in_body,
        grid=(NB // BM,),
        in_specs=[
            pl.BlockSpec((BM, FB), lambda i: (i, 0)),
            pl.BlockSpec((FB, HP), lambda i: (0, 0)),
        ],
        out_specs=[
            pl.BlockSpec((BM, HP), lambda i: (i, 0)),
            pl.BlockSpec((BM, HP), lambda i: (i, 0)),
        ],
        out_shape=[
            jax.ShapeDtypeStruct((NB, HP), jnp.bfloat16),
            jax.ShapeDtypeStruct((NB, HP), jnp.float32),
        ],
    )(f_bonds, w_i_t)


def _mm_h_body(t_ref, w_ref, inp_ref, out_ref):
    x = jnp.dot(t_ref[...].astype(jnp.bfloat16), w_ref[...],
                preferred_element_type=jnp.float32)
    out_ref[...] = jnp.maximum(inp_ref[...].astype(jnp.float32) + x, 0.0)


def _tc_h(t, w_h_t, inp):
    return pl.pallas_call(
        _mm_h_body,
        grid=(NB // BM,),
        in_specs=[
            pl.BlockSpec((BM, HP), lambda i: (i, 0)),
            pl.BlockSpec((HP, HP), lambda i: (0, 0)),
            pl.BlockSpec((BM, HP), lambda i: (i, 0)),
        ],
        out_specs=pl.BlockSpec((BM, HP), lambda i: (i, 0)),
        out_shape=jax.ShapeDtypeStruct((NB, HP), jnp.float32),
    )(t, w_h_t, inp)


def _readout_body(fa_ref, am_ref, woa_ref, woh_ref, bo_ref,
                  w1_ref, b1_ref, w2_ref, b2_ref, wr_ref, br_ref, out_ref):
    x = jnp.dot(fa_ref[...], woa_ref[...], preferred_element_type=jnp.float32)
    x = x + jnp.dot(am_ref[...][:, :H], woh_ref[...],
                    preferred_element_type=jnp.float32)
    x = jnp.maximum(x + bo_ref[...], 0.0)
    h = jnp.maximum(
        jnp.dot(x, w1_ref[...], preferred_element_type=jnp.float32)
        + b1_ref[...], 0.0)
    h = jnp.maximum(
        jnp.dot(h, w2_ref[...], preferred_element_type=jnp.float32)
        + b2_ref[...], 0.0)
    o = jnp.dot(h, wr_ref[...], preferred_element_type=jnp.float32) + br_ref[...]
    out_ref[...] = jax.nn.sigmoid(o)


def _tc_readout(f_atoms, am, woa_t, woh_t, b_o, w1_t, b1, w2_t, b2, wr_t, br):
    return pl.pallas_call(
        _readout_body,
        grid=(NA // BMA,),
        in_specs=[
            pl.BlockSpec((BMA, FA), lambda i: (i, 0)),
            pl.BlockSpec((BMA, HP), lambda i: (i, 0)),
            pl.BlockSpec((FA, H), lambda i: (0, 0)),
            pl.BlockSpec((H, H), lambda i: (0, 0)),
            pl.BlockSpec((1, H), lambda i: (0, 0)),
            pl.BlockSpec((H, H), lambda i: (0, 0)),
            pl.BlockSpec((1, H), lambda i: (0, 0)),
            pl.BlockSpec((H, H), lambda i: (0, 0)),
            pl.BlockSpec((1, H), lambda i: (0, 0)),
            pl.BlockSpec((H, 1), lambda i: (0, 0)),
            pl.BlockSpec((1, 1), lambda i: (0, 0)),
        ],
        out_specs=pl.BlockSpec((BMA, 1), lambda i: (i, 0)),
        out_shape=jax.ShapeDtypeStruct((NA, 1), jnp.float32),
    )(f_atoms, am, woa_t, woh_t, b_o, w1_t, b1, w2_t, b2, wr_t, br)


def kernel(f_atoms, f_bonds, a2b, b2a, b2revb,
           W_i, W_h, W_o, b_o, W_f1, b_f1, W_f2, b_f2, W_r, b_r):
    w_i_t = jnp.zeros((FB, HP), jnp.bfloat16).at[:, :H].set(
        W_i.T.astype(jnp.bfloat16))
    w_h_t = jnp.zeros((HP, HP), jnp.bfloat16).at[:H, :H].set(
        W_h.T.astype(jnp.bfloat16))
    a2b_flat = jnp.pad(a2b, ((0, NA_PAD - NA), (0, 0))).reshape(-1)
    idx2 = jnp.concatenate([b2a.reshape(NB // GD_CB, GD_CB),
                            b2revb.reshape(NB // GD_CB, GD_CB)],
                           axis=1).reshape(-1)

    inp, msg = _tc_in(f_bonds, w_i_t)
    for _ in range(DEPTH - 1):
        am = _sc_segsum(msg, a2b_flat)
        t = _sc_gather_diff(am, msg, idx2)
        msg = _tc_h(t, w_h_t, inp)
    am = _sc_segsum(msg, a2b_flat)

    out = _tc_readout(
        f_atoms, am,
        W_o[:, :FA].T, W_o[:, FA:].T, b_o.reshape(1, H),
        W_f1.T, b_f1.reshape(1, H), W_f2.T, b_f2.reshape(1, H),
        W_r.T, b_r.reshape(1, 1))
    return out[1:]


# speedup vs baseline: 1.2152x; 1.0490x over previous
import functools

import jax
import jax.numpy as jnp
from jax import lax
from jax.experimental import pallas as pl
from jax.experimental.pallas import tpu as pltpu
from jax.experimental.pallas import tpu_sc as plsc

NC, NS, L = 2, 16, 16
NW = NC * NS

NA = 10000
NB = 160000
MAXNB = 16
H = 600
HP = 640
FA = 133
FB = 147
DEPTH = 6

NA_PAD = 10240
A_PER_W = NA_PAD // NW
SEG_CA = 4
B_PER_W = NB // NW
GD_CB = 40

_mesh = plsc.VectorSubcoreMesh(core_axis_name="c", subcore_axis_name="s")


SEG_NCH = A_PER_W // SEG_CA
GD_NCH = B_PER_W // GD_CB


@functools.partial(
    pl.kernel,
    out_type=jax.ShapeDtypeStruct((NA_PAD, HP), jnp.float32),
    mesh=_mesh,
    scratch_types=[
        pltpu.VMEM((A_PER_W * MAXNB,), jnp.int32),
        pltpu.VMEM((SEG_CA * MAXNB, HP), jnp.float32),
        pltpu.VMEM((SEG_CA * MAXNB, HP), jnp.float32),
        pltpu.VMEM((SEG_CA, HP), jnp.float32),
        pltpu.VMEM((SEG_CA, HP), jnp.float32),
        pltpu.SemaphoreType.DMA,
        pltpu.SemaphoreType.DMA,
        pltpu.SemaphoreType.DMA,
        pltpu.SemaphoreType.DMA,
    ],
)
def _sc_segsum(msg_hbm, idx_hbm, out_hbm, idx_all, rows0, rows1,
               acc0, acc1, semg0, semg1, sems0, sems1):
    wid = lax.axis_index("s") * NC + lax.axis_index("c")
    rows = (rows0, rows1)
    accs, semgs, semss = (acc0, acc1), (semg0, semg1), (sems0, sems1)
    pltpu.sync_copy(idx_hbm.at[pl.ds(wid * A_PER_W * MAXNB, A_PER_W * MAXNB)],
                    idx_all)

    def idx_at(ci):
        return idx_all.at[pl.ds(ci * SEG_CA * MAXNB, SEG_CA * MAXNB)]

    def fire(ci, p):
        pltpu.async_copy(msg_hbm.at[idx_at(ci)], rows[p], semgs[p])

    def out_at(ci):
        return out_hbm.at[pl.ds(wid * A_PER_W + ci * SEG_CA, SEG_CA)]

    def consume(ci, p):
        pltpu.make_async_copy(msg_hbm.at[idx_at(ci)], rows[p], semgs[p]).wait()

        @pl.when(ci >= 2)
        def _():
            wait_store(ci - 2, p)

        rp, ac = rows[p], accs[p]

        def atom_body(a, carry):
            base = a * MAXNB
            for c in range(HP // L):
                vals = [rp[base + j, pl.ds(c * L, L)] for j in range(MAXNB)]
                while len(vals) > 1:
                    vals = [vals[i] + vals[i + 1]
                            for i in range(0, len(vals), 2)]
                ac[a, pl.ds(c * L, L)] = vals[0]
            return carry

        lax.fori_loop(0, SEG_CA, atom_body, 0)
        pltpu.async_copy(ac, out_at(ci), semss[p])

    def wait_store(ci, p):
        pltpu.make_async_copy(accs[p], out_at(ci), semss[p]).wait()

    fire(0, 0)
    fire(1, 1)

    def body(cj, carry):
        ci = 2 * cj
        consume(ci, 0)
        fire(ci + 2, 0)
        consume(ci + 1, 1)
        fire(ci + 3, 1)
        return carry

    lax.fori_loop(0, SEG_NCH // 2 - 1, body, 0)
    consume(SEG_NCH - 2, 0)
    consume(SEG_NCH - 1, 1)
    wait_store(SEG_NCH - 2, 0)
    wait_store(SEG_NCH - 1, 1)


@functools.partial(
    pl.kernel,
    out_type=jax.ShapeDtypeStruct((NB, HP), jnp.float32),
    mesh=_mesh,
    scratch_types=[
        pltpu.VMEM((GD_NCH * 2 * GD_CB,), jnp.int32),
        pltpu.VMEM((GD_CB, HP), jnp.float32),
        pltpu.VMEM((GD_CB, HP), jnp.float32),
        pltpu.VMEM((GD_CB, HP), jnp.float32),
        pltpu.VMEM((GD_CB, HP), jnp.float32),
        pltpu.SemaphoreType.DMA,
        pltpu.SemaphoreType.DMA,
        pltpu.SemaphoreType.DMA,
        pltpu.SemaphoreType.DMA,
        pltpu.SemaphoreType.DMA,
        pltpu.SemaphoreType.DMA,
    ],
)
def _sc_gather_diff(am_hbm, msg_hbm, idx2_hbm, out_hbm,
                    ii_all, arow0, arow1, mrow0, mrow1,
                    semA0, semA1, semM0, semM1, semS0, semS1):
    wid = lax.axis_index("s") * NC + lax.axis_index("c")
    arows, mrows = (arow0, arow1), (mrow0, mrow1)
    semsA, semsM = (semA0, semA1), (semM0, semM1)
    semsS = (semS0, semS1)
    pltpu.sync_copy(
        idx2_hbm.at[pl.ds(wid * GD_NCH * 2 * GD_CB, GD_NCH * 2 * GD_CB)],
        ii_all)

    def idxA_at(ci):
        return ii_all.at[pl.ds(ci * 2 * GD_CB, GD_CB)]

    def idxM_at(ci):
        return ii_all.at[pl.ds(ci * 2 * GD_CB + GD_CB, GD_CB)]

    def fire(ci, p):
        pltpu.async_copy(am_hbm.at[idxA_at(ci)], arows[p], semsA[p])
        pltpu.async_copy(msg_hbm.at[idxM_at(ci)], mrows[p], semsM[p])

    def out_at(ci):
        return out_hbm.at[pl.ds(wid * B_PER_W + ci * GD_CB, GD_CB)]

    def consume(ci, p):
        pltpu.make_async_copy(am_hbm.at[idxA_at(ci)],
                              arows[p], semsA[p]).wait()
        pltpu.make_async_copy(msg_hbm.at[idxM_at(ci)],
                              mrows[p], semsM[p]).wait()
        ap, mp = arows[p], mrows[p]

        def row_body(r, carry2):
            for c in range(HP // L):
                ap[r, pl.ds(c * L, L)] = (
                    ap[r, pl.ds(c * L, L)] - mp[r, pl.ds(c * L, L)])
            return carry2

        lax.fori_loop(0, GD_CB, row_body, 0)
        pltpu.sync_copy(ap, out_at(ci))

    fire(0, 0)
    fire(1, 1)

    def body(cj, carry):
        ci = 2 * cj
        consume(ci, 0)
        fire(ci + 2, 0)
        consume(ci + 1, 1)
        fire(ci + 3, 1)
        return carry

    lax.fori_loop(0, (GD_NCH - 1) // 2 - 1, body, 0)
    consume(GD_NCH - 3, 0)
    fire(GD_NCH - 1, 0)
    consume(GD_NCH - 2, 1)
    consume(GD_NCH - 1, 0)


BM = 1600
BMA = 400


def _mm_in_body(fb_ref, w_ref, inp_ref, msg_ref):
    x = jnp.dot(fb_ref[...].astype(jnp.bfloat16), w_ref[...],
                preferred_element_type=jnp.float32)
    inp_ref[...] = x.astype(jnp.bfloat16)
    msg_ref[...] = jnp.maximum(x, 0.0)


def _tc_in(f_bonds, w_i_t):
    return pl.pallas_call(
        _mm_in_body,
        grid=(NB // BM,),
        in_specs=[
            pl.BlockSpec((BM, FB), lambda i: (i, 0)),
            pl.BlockSpec((FB, HP), lambda i: (0, 0)),
        ],
        out_specs=[
            pl.BlockSpec((BM, HP), lambda i: (i, 0)),
            pl.BlockSpec((BM, HP), lambda i: (i, 0)),
        ],
        out_shape=[
            jax.ShapeDtypeStruct((NB, HP), jnp.bfloat16),
            jax.ShapeDtypeStruct((NB, HP), jnp.float32),
        ],
    )(f_bonds, w_i_t)


def _mm_h_body(t_ref, w_ref, inp_ref, out_ref):
    x = jnp.dot(t_ref[...].astype(jnp.bfloat16), w_ref[...],
                preferred_element_type=jnp.float32)
    out_ref[...] = jnp.maximum(inp_ref[...].astype(jnp.float32) + x, 0.0)


def _tc_h(t, w_h_t, inp):
    return pl.pallas_call(
        _mm_h_body,
        grid=(NB // BM,),
        in_specs=[
            pl.BlockSpec((BM, HP), lambda i: (i, 0)),
            pl.BlockSpec((HP, HP), lambda i: (0, 0)),
            pl.BlockSpec((BM, HP), lambda i: (i, 0)),
        ],
        out_specs=pl.BlockSpec((BM, HP), lambda i: (i, 0)),
        out_shape=jax.ShapeDtypeStruct((NB, HP), jnp.float32),
    )(t, w_h_t, inp)


def _readout_body(fa_ref, am_ref, woa_ref, woh_ref, bo_ref,
                  w1_ref, b1_ref, w2_ref, b2_ref, wr_ref, br_ref, out_ref):
    x = jnp.dot(fa_ref[...], woa_ref[...], preferred_element_type=jnp.float32)
    x = x + jnp.dot(am_ref[...][:, :H], woh_ref[...],
                    preferred_element_type=jnp.float32)
    x = jnp.maximum(x + bo_ref[...], 0.0)
    h = jnp.maximum(
        jnp.dot(x, w1_ref[...], preferred_element_type=jnp.float32)
        + b1_ref[...], 0.0)
    h = jnp.maximum(
        jnp.dot(h, w2_ref[...], preferred_element_type=jnp.float32)
        + b2_ref[...], 0.0)
    o = jnp.dot(h, wr_ref[...], preferred_element_type=jnp.float32) + br_ref[...]
    out_ref[...] = jax.nn.sigmoid(o)


def _tc_readout(f_atoms, am, woa_t, woh_t, b_o, w1_t, b1, w2_t, b2, wr_t, br):
    return pl.pallas_call(
        _readout_body,
        grid=(NA // BMA,),
        in_specs=[
            pl.BlockSpec((BMA, FA), lambda i: (i, 0)),
            pl.BlockSpec((BMA, HP), lambda i: (i, 0)),
            pl.BlockSpec((FA, H), lambda i: (0, 0)),
            pl.BlockSpec((H, H), lambda i: (0, 0)),
            pl.BlockSpec((1, H), lambda i: (0, 0)),
            pl.BlockSpec((H, H), lambda i: (0, 0)),
            pl.BlockSpec((1, H), lambda i: (0, 0)),
            pl.BlockSpec((H, H), lambda i: (0, 0)),
            pl.BlockSpec((1, H), lambda i: (0, 0)),
            pl.BlockSpec((H, 1), lambda i: (0, 0)),
            pl.BlockSpec((1, 1), lambda i: (0, 0)),
        ],
        out_specs=pl.BlockSpec((BMA, 1), lambda i: (i, 0)),
        out_shape=jax.ShapeDtypeStruct((NA, 1), jnp.float32),
    )(f_atoms, am, woa_t, woh_t, b_o, w1_t, b1, w2_t, b2, wr_t, br)


def kernel(f_atoms, f_bonds, a2b, b2a, b2revb,
           W_i, W_h, W_o, b_o, W_f1, b_f1, W_f2, b_f2, W_r, b_r):
    w_i_t = jnp.zeros((FB, HP), jnp.bfloat16).at[:, :H].set(
        W_i.T.astype(jnp.bfloat16))
    w_h_t = jnp.zeros((HP, HP), jnp.bfloat16).at[:H, :H].set(
        W_h.T.astype(jnp.bfloat16))
    a2b_flat = jnp.pad(a2b, ((0, NA_PAD - NA), (0, 0))).reshape(-1)
    idx2 = jnp.concatenate([b2a.reshape(NB // GD_CB, GD_CB),
                            b2revb.reshape(NB // GD_CB, GD_CB)],
                           axis=1).reshape(-1)

    inp, msg = _tc_in(f_bonds, w_i_t)
    for _ in range(DEPTH - 1):
        am = _sc_segsum(msg, a2b_flat)
        t = _sc_gather_diff(am, msg, idx2)
        msg = _tc_h(t, w_h_t, inp)
    am = _sc_segsum(msg, a2b_flat)

    out = _tc_readout(
        f_atoms, am,
        W_o[:, :FA].T, W_o[:, FA:].T, b_o.reshape(1, H),
        W_f1.T, b_f1.reshape(1, H), W_f2.T, b_f2.reshape(1, H),
        W_r.T, b_r.reshape(1, 1))
    return out[1:]


# speedup vs baseline: 1.2216x; 1.0053x over previous
import functools

import jax
import jax.numpy as jnp
from jax import lax
from jax.experimental import pallas as pl
from jax.experimental.pallas import tpu as pltpu
from jax.experimental.pallas import tpu_sc as plsc

NC, NS, L = 2, 16, 16
NW = NC * NS

NA = 10000
NB = 160000
MAXNB = 16
H = 600
HP = 640
FA = 133
FB = 147
DEPTH = 6

NA_PAD = 10240
A_PER_W = NA_PAD // NW
SEG_CA = 4
B_PER_W = NB // NW
GD_CB = 40

_mesh = plsc.VectorSubcoreMesh(core_axis_name="c", subcore_axis_name="s")


SEG_NCH = A_PER_W // SEG_CA
GD_NCH = B_PER_W // GD_CB


@functools.partial(
    pl.kernel,
    out_type=jax.ShapeDtypeStruct((NA_PAD, HP), jnp.float32),
    mesh=_mesh,
    scratch_types=[
        pltpu.VMEM((A_PER_W * MAXNB,), jnp.int32),
        pltpu.VMEM((SEG_CA * MAXNB, HP), jnp.float32),
        pltpu.VMEM((SEG_CA * MAXNB, HP), jnp.float32),
        pltpu.VMEM((SEG_CA, HP), jnp.float32),
        pltpu.VMEM((SEG_CA, HP), jnp.float32),
        pltpu.SemaphoreType.DMA,
        pltpu.SemaphoreType.DMA,
        pltpu.SemaphoreType.DMA,
        pltpu.SemaphoreType.DMA,
    ],
)
def _sc_segsum(msg_hbm, idx_hbm, out_hbm, idx_all, rows0, rows1,
               acc0, acc1, semg0, semg1, sems0, sems1):
    wid = lax.axis_index("s") * NC + lax.axis_index("c")
    rows = (rows0, rows1)
    accs, semgs, semss = (acc0, acc1), (semg0, semg1), (sems0, sems1)
    pltpu.sync_copy(idx_hbm.at[pl.ds(wid * A_PER_W * MAXNB, A_PER_W * MAXNB)],
                    idx_all)

    def idx_at(ci):
        return idx_all.at[pl.ds(ci * SEG_CA * MAXNB, SEG_CA * MAXNB)]

    def fire(ci, p):
        pltpu.async_copy(msg_hbm.at[idx_at(ci)], rows[p], semgs[p])

    def out_at(ci):
        return out_hbm.at[pl.ds(wid * A_PER_W + ci * SEG_CA, SEG_CA)]

    def consume(ci, p):
        pltpu.make_async_copy(msg_hbm.at[idx_at(ci)], rows[p], semgs[p]).wait()

        @pl.when(ci >= 2)
        def _():
            wait_store(ci - 2, p)

        rp, ac = rows[p], accs[p]

        def atom_body(a, carry):
            base = a * MAXNB
            for c in range(HP // L):
                vals = [rp[base + j, pl.ds(c * L, L)] for j in range(MAXNB)]
                while len(vals) > 1:
                    vals = [vals[i] + vals[i + 1]
                            for i in range(0, len(vals), 2)]
                ac[a, pl.ds(c * L, L)] = vals[0]
            return carry

        lax.fori_loop(0, SEG_CA, atom_body, 0)
        pltpu.async_copy(ac, out_at(ci), semss[p])

    def wait_store(ci, p):
        pltpu.make_async_copy(accs[p], out_at(ci), semss[p]).wait()

    fire(0, 0)
    fire(1, 1)

    def body(cj, carry):
        ci = 2 * cj
        consume(ci, 0)
        fire(ci + 2, 0)
        consume(ci + 1, 1)
        fire(ci + 3, 1)
        return carry

    lax.fori_loop(0, SEG_NCH // 2 - 1, body, 0)
    consume(SEG_NCH - 2, 0)
    consume(SEG_NCH - 1, 1)
    wait_store(SEG_NCH - 2, 0)
    wait_store(SEG_NCH - 1, 1)


@functools.partial(
    pl.kernel,
    out_type=jax.ShapeDtypeStruct((NB, HP), jnp.float32),
    mesh=_mesh,
    scratch_types=[
        pltpu.VMEM((GD_NCH * 2 * GD_CB,), jnp.int32),
        pltpu.VMEM((GD_CB, HP), jnp.float32),
        pltpu.VMEM((GD_CB, HP), jnp.float32),
        pltpu.VMEM((GD_CB, HP), jnp.float32),
        pltpu.VMEM((GD_CB, HP), jnp.float32),
        pltpu.SemaphoreType.DMA,
        pltpu.SemaphoreType.DMA,
        pltpu.SemaphoreType.DMA,
        pltpu.SemaphoreType.DMA,
        pltpu.SemaphoreType.DMA,
        pltpu.SemaphoreType.DMA,
    ],
)
def _sc_gather_diff(am_hbm, msg_hbm, idx2_hbm, out_hbm,
                    ii_all, arow0, arow1, mrow0, mrow1,
                    semA0, semA1, semM0, semM1, semS0, semS1):
    wid = lax.axis_index("s") * NC + lax.axis_index("c")
    arows, mrows = (arow0, arow1), (mrow0, mrow1)
    semsA, semsM = (semA0, semA1), (semM0, semM1)
    semsS = (semS0, semS1)
    pltpu.sync_copy(
        idx2_hbm.at[pl.ds(wid * GD_NCH * 2 * GD_CB, GD_NCH * 2 * GD_CB)],
        ii_all)

    def idxA_at(ci):
        return ii_all.at[pl.ds(ci * 2 * GD_CB, GD_CB)]

    def idxM_at(ci):
        return ii_all.at[pl.ds(ci * 2 * GD_CB + GD_CB, GD_CB)]

    def fire(ci, p):
        pltpu.async_copy(am_hbm.at[idxA_at(ci)], arows[p], semsA[p])
        pltpu.async_copy(msg_hbm.at[idxM_at(ci)], mrows[p], semsM[p])

    def out_at(ci):
        return out_hbm.at[pl.ds(wid * B_PER_W + ci * GD_CB, GD_CB)]

    def consume(ci, p):
        pltpu.make_async_copy(am_hbm.at[idxA_at(ci)],
                              arows[p], semsA[p]).wait()
        pltpu.make_async_copy(msg_hbm.at[idxM_at(ci)],
                              mrows[p], semsM[p]).wait()
        ap, mp = arows[p], mrows[p]

        def row_body(r, carry2):
            for c in range(HP // L):
                ap[r, pl.ds(c * L, L)] = (
                    ap[r, pl.ds(c * L, L)] - mp[r, pl.ds(c * L, L)])
            return carry2

        lax.fori_loop(0, GD_CB, row_body, 0)
        pltpu.sync_copy(ap, out_at(ci))

    fire(0, 0)
    fire(1, 1)

    def body(cj, carry):
        ci = 2 * cj
        consume(ci, 0)
        fire(ci + 2, 0)
        consume(ci + 1, 1)
        fire(ci + 3, 1)
        return carry

    lax.fori_loop(0, (GD_NCH - 1) // 2 - 1, body, 0)
    consume(GD_NCH - 3, 0)
    fire(GD_NCH - 1, 0)
    consume(GD_NCH - 2, 1)
    consume(GD_NCH - 1, 0)


BM = 3200
BMA = 400


def _mm_in_body(fb_ref, w_ref, inp_ref, msg_ref):
    x = jnp.dot(fb_ref[...].astype(jnp.bfloat16), w_ref[...],
                preferred_element_type=jnp.float32)
    inp_ref[...] = x.astype(jnp.bfloat16)
    msg_ref[...] = jnp.maximum(x, 0.0)


def _tc_in(f_bonds, w_i_t):
    return pl.pallas_call(
        _mm_in_body,
        grid=(NB // BM,),
        in_specs=[
            pl.BlockSpec((BM, FB), lambda i: (i, 0)),
            pl.BlockSpec((FB, HP), lambda i: (0, 0)),
        ],
        out_specs=[
            pl.BlockSpec((BM, HP), lambda i: (i, 0)),
            pl.BlockSpec((BM, HP), lambda i: (i, 0)),
        ],
        out_shape=[
            jax.ShapeDtypeStruct((NB, HP), jnp.bfloat16),
            jax.ShapeDtypeStruct((NB, HP), jnp.float32),
        ],
    )(f_bonds, w_i_t)


def _mm_h_body(t_ref, w_ref, inp_ref, out_ref):
    x = jnp.dot(t_ref[...].astype(jnp.bfloat16), w_ref[...],
                preferred_element_type=jnp.float32)
    out_ref[...] = jnp.maximum(inp_ref[...].astype(jnp.float32) + x, 0.0)


def _tc_h(t, w_h_t, inp):
    return pl.pallas_call(
        _mm_h_body,
        grid=(NB // BM,),
        in_specs=[
            pl.BlockSpec((BM, HP), lambda i: (i, 0)),
            pl.BlockSpec((HP, HP), lambda i: (0, 0)),
            pl.BlockSpec((BM, HP), lambda i: (i, 0)),
        ],
        out_specs=pl.BlockSpec((BM, HP), lambda i: (i, 0)),
        out_shape=jax.ShapeDtypeStruct((NB, HP), jnp.float32),
    )(t, w_h_t, inp)


def _readout_body(fa_ref, am_ref, woa_ref, woh_ref, bo_ref,
                  w1_ref, b1_ref, w2_ref, b2_ref, wr_ref, br_ref, out_ref):
    x = jnp.dot(fa_ref[...], woa_ref[...], preferred_element_type=jnp.float32)
    x = x + jnp.dot(am_ref[...][:, :H], woh_ref[...],
                    preferred_element_type=jnp.float32)
    x = jnp.maximum(x + bo_ref[...], 0.0)
    h = jnp.maximum(
        jnp.dot(x, w1_ref[...], preferred_element_type=jnp.float32)
        + b1_ref[...], 0.0)
    h = jnp.maximum(
        jnp.dot(h, w2_ref[...], preferred_element_type=jnp.float32)
        + b2_ref[...], 0.0)
    o = jnp.dot(h, wr_ref[...], preferred_element_type=jnp.float32) + br_ref[...]
    out_ref[...] = jax.nn.sigmoid(o)


def _tc_readout(f_atoms, am, woa_t, woh_t, b_o, w1_t, b1, w2_t, b2, wr_t, br):
    return pl.pallas_call(
        _readout_body,
        grid=(NA // BMA,),
        in_specs=[
            pl.BlockSpec((BMA, FA), lambda i: (i, 0)),
            pl.BlockSpec((BMA, HP), lambda i: (i, 0)),
            pl.BlockSpec((FA, H), lambda i: (0, 0)),
            pl.BlockSpec((H, H), lambda i: (0, 0)),
            pl.BlockSpec((1, H), lambda i: (0, 0)),
            pl.BlockSpec((H, H), lambda i: (0, 0)),
            pl.BlockSpec((1, H), lambda i: (0, 0)),
            pl.BlockSpec((H, H), lambda i: (0, 0)),
            pl.BlockSpec((1, H), lambda i: (0, 0)),
            pl.BlockSpec((H, 1), lambda i: (0, 0)),
            pl.BlockSpec((1, 1), lambda i: (0, 0)),
        ],
        out_specs=pl.BlockSpec((BMA, 1), lambda i: (i, 0)),
        out_shape=jax.ShapeDtypeStruct((NA, 1), jnp.float32),
    )(f_atoms, am, woa_t, woh_t, b_o, w1_t, b1, w2_t, b2, wr_t, br)


def kernel(f_atoms, f_bonds, a2b, b2a, b2revb,
           W_i, W_h, W_o, b_o, W_f1, b_f1, W_f2, b_f2, W_r, b_r):
    w_i_t = jnp.zeros((FB, HP), jnp.bfloat16).at[:, :H].set(
        W_i.T.astype(jnp.bfloat16))
    w_h_t = jnp.zeros((HP, HP), jnp.bfloat16).at[:H, :H].set(
        W_h.T.astype(jnp.bfloat16))
    a2b_flat = jnp.pad(a2b, ((0, NA_PAD - NA), (0, 0))).reshape(-1)
    idx2 = jnp.concatenate([b2a.reshape(NB // GD_CB, GD_CB),
                            b2revb.reshape(NB // GD_CB, GD_CB)],
                           axis=1).reshape(-1)

    inp, msg = _tc_in(f_bonds, w_i_t)
    for _ in range(DEPTH - 1):
        am = _sc_segsum(msg, a2b_flat)
        t = _sc_gather_diff(am, msg, idx2)
        msg = _tc_h(t, w_h_t, inp)
    am = _sc_segsum(msg, a2b_flat)

    out = _tc_readout(
        f_atoms, am,
        W_o[:, :FA].T, W_o[:, FA:].T, b_o.reshape(1, H),
        W_f1.T, b_f1.reshape(1, H), W_f2.T, b_f2.reshape(1, H),
        W_r.T, b_r.reshape(1, 1))
    return out[1:]
